# Initial kernel scaffold; baseline (speedup 1.0000x reference)
import jax
import jax.numpy as jnp
from jax.experimental import pallas as pl


def kernel(input_ids, weight):
    raise NotImplementedError("write your pallas kernel here")


# speedup vs baseline: 1.2910x; 1.2910x over previous
import jax
import jax.numpy as jnp
from jax import lax
from jax.experimental import pallas as pl
from jax.experimental.pallas import tpu as pltpu
from jax.experimental.pallas import tpu_sc as plsc

_V = 1000
_D = 128
_P = 8
_EPS = 0.01
_B = 1024
_L = 50

_info = plsc.get_sparse_core_info()
_NC = _info.num_cores
_NS = _info.num_subcores
_NW = _NC * _NS

_ROWS = _P * _B * _L
_RPW = _ROWS // _NW
_GR = 128
_BLOCKS = _RPW // _GR


def _build_table_body(w_ref, mu_ref, out_ref):
    out_ref[...] = w_ref[...][None, :, :] + _EPS * mu_ref[...]


def _gather_body(table_ref, idx_ref, out_ref, idx_v, rows_v, sem):
    c = lax.axis_index("c")
    s = lax.axis_index("s")
    wid = s * _NC + c
    pltpu.sync_copy(idx_ref.at[wid], idx_v)
    base = wid * _RPW

    def step(k, carry):
        pltpu.async_copy(table_ref.at[idx_v.at[k]], rows_v, sem).wait()
        pltpu.sync_copy(rows_v, out_ref.at[pl.ds(base + k * _GR, _GR)])
        return carry

    lax.fori_loop(0, _BLOCKS, step, 0)


def kernel(input_ids, weight):
    mu_key = jax.random.key(42)
    mu = jax.random.randint(mu_key, (_P, _V, _D), 0, 2).astype(jnp.float32) * 2.0 - 1.0

    table = pl.pallas_call(
        _build_table_body,
        out_shape=jax.ShapeDtypeStruct((_P, _V, _D), jnp.float32),
    )(weight, mu)
    table2d = table.reshape(_P * _V, _D)

    ids_flat = input_ids.reshape(-1).astype(jnp.int32)
    gidx = (ids_flat[None, :] + _V * jnp.arange(_P, dtype=jnp.int32)[:, None])
    gidx = gidx.reshape(_NW, _BLOCKS, _GR)

    out_flat = pl.kernel(
        _gather_body,
        out_type=jax.ShapeDtypeStruct((_ROWS, _D), jnp.float32),
        mesh=plsc.VectorSubcoreMesh(core_axis_name="c", subcore_axis_name="s"),
        scratch_types=[
            pltpu.VMEM((_BLOCKS, _GR), jnp.int32),
            pltpu.VMEM((_GR, _D), jnp.float32),
            pltpu.SemaphoreType.DMA,
        ],
    )(table2d, gidx)

    return out_flat.reshape(_P, _B, _L, _D)


# speedup vs baseline: 1.3668x; 1.0588x over previous
import jax
import jax.numpy as jnp
from jax import lax
from jax.experimental import pallas as pl
from jax.experimental.pallas import tpu as pltpu
from jax.experimental.pallas import tpu_sc as plsc

_V = 1000
_D = 128
_P = 8
_EPS = 0.01
_B = 1024
_L = 50

_info = plsc.get_sparse_core_info()
_NC = _info.num_cores
_NS = _info.num_subcores
_NW = _NC * _NS

_ROWS = _P * _B * _L
_RPW = _ROWS // _NW
_GR = 128
_BLOCKS = _RPW // _GR


def _build_table_body(w_ref, mu_ref, out_ref):
    out_ref[...] = w_ref[...][None, :, :] + _EPS * mu_ref[...]


def _gather_body(table_ref, idx_ref, out_ref, idx_v, rows0, rows1, g0, g1, w0, w1):
    c = lax.axis_index("c")
    s = lax.axis_index("s")
    wid = s * _NC + c
    pltpu.sync_copy(idx_ref.at[wid], idx_v)
    base = wid * _RPW

    def gstart(j, buf, sem):
        pltpu.async_copy(table_ref.at[idx_v.at[j]], buf, sem)

    def gwait(j, buf, sem):
        pltpu.make_async_copy(table_ref.at[idx_v.at[j]], buf, sem).wait()

    def wstart(j, buf, sem):
        pltpu.async_copy(buf, out_ref.at[pl.ds(base + j * _GR, _GR)], sem)

    def wwait(j, buf, sem):
        pltpu.make_async_copy(buf, out_ref.at[pl.ds(base + j * _GR, _GR)], sem).wait()

    gstart(0, rows0, g0)

    def step(k, carry):
        j0 = 2 * k
        j1 = j0 + 1
        gwait(j0, rows0, g0)
        wstart(j0, rows0, w0)

        @pl.when(k > 0)
        def _():
            wwait(j0 - 1, rows1, w1)
        gstart(j1, rows1, g1)

        gwait(j1, rows1, g1)
        wstart(j1, rows1, w1)
        wwait(j0, rows0, w0)

        @pl.when(k < _BLOCKS // 2 - 1)
        def _():
            gstart(j1 + 1, rows0, g0)
        return carry

    lax.fori_loop(0, _BLOCKS // 2, step, 0)
    wwait(_BLOCKS - 1, rows1, w1)


def kernel(input_ids, weight):
    mu_key = jax.random.key(42)
    mu = jax.random.randint(mu_key, (_P, _V, _D), 0, 2).astype(jnp.float32) * 2.0 - 1.0

    table = pl.pallas_call(
        _build_table_body,
        out_shape=jax.ShapeDtypeStruct((_P, _V, _D), jnp.float32),
    )(weight, mu)
    table2d = table.reshape(_P * _V, _D)

    ids_flat = input_ids.reshape(-1).astype(jnp.int32)
    gidx = (ids_flat[None, :] + _V * jnp.arange(_P, dtype=jnp.int32)[:, None])
    gidx = gidx.reshape(_NW, _BLOCKS, _GR)

    out_flat = pl.kernel(
        _gather_body,
        out_type=jax.ShapeDtypeStruct((_ROWS, _D), jnp.float32),
        mesh=plsc.VectorSubcoreMesh(core_axis_name="c", subcore_axis_name="s"),
        scratch_types=[
            pltpu.VMEM((_BLOCKS, _GR), jnp.int32),
            pltpu.VMEM((_GR, _D), jnp.float32),
            pltpu.VMEM((_GR, _D), jnp.float32),
            pltpu.SemaphoreType.DMA,
            pltpu.SemaphoreType.DMA,
            pltpu.SemaphoreType.DMA,
            pltpu.SemaphoreType.DMA,
        ],
    )(table2d, gidx)

    return out_flat.reshape(_P, _B, _L, _D)


# speedup vs baseline: 1.7809x; 1.3029x over previous
import jax
import jax.numpy as jnp
from jax import lax
from jax.experimental import pallas as pl
from jax.experimental.pallas import tpu as pltpu
from jax.experimental.pallas import tpu_sc as plsc

_V = 1000
_D = 128
_P = 8
_EPS = 0.01
_B = 1024
_L = 50

_info = plsc.get_sparse_core_info()
_NC = _info.num_cores
_NS = _info.num_subcores
_NW = _NC * _NS

_PAIRS = _P * _B
_PPW = _PAIRS // _NW


def _build_table_body(w_ref, mu_ref, out_ref):
    out_ref[...] = w_ref[...][None, :, :] + _EPS * mu_ref[...]


def _gather_body(table_ref, idx_ref, out_ref, idx_v, rows0, rows1, g0, g1, w0, w1):
    c = lax.axis_index("c")
    s = lax.axis_index("s")
    wid = s * _NC + c
    pltpu.sync_copy(idx_ref.at[wid], idx_v)
    p = wid // (_B // _PPW)
    b0 = (wid % (_B // _PPW)) * _PPW

    def gstart(j, buf, sem):
        pltpu.async_copy(table_ref.at[idx_v.at[j]], buf, sem)

    def gwait(j, buf, sem):
        pltpu.make_async_copy(table_ref.at[idx_v.at[j]], buf, sem).wait()

    def wstart(j, buf, sem):
        pltpu.async_copy(buf, out_ref.at[p, b0 + j], sem)

    def wwait(j, buf, sem):
        pltpu.make_async_copy(buf, out_ref.at[p, b0 + j], sem).wait()

    gstart(0, rows0, g0)

    def step(k, carry):
        j0 = 2 * k
        j1 = j0 + 1
        gwait(j0, rows0, g0)
        wstart(j0, rows0, w0)

        @pl.when(k > 0)
        def _():
            wwait(j0 - 1, rows1, w1)
        gstart(j1, rows1, g1)

        gwait(j1, rows1, g1)
        wstart(j1, rows1, w1)
        wwait(j0, rows0, w0)

        @pl.when(k < _PPW // 2 - 1)
        def _():
            gstart(j1 + 1, rows0, g0)
        return carry

    lax.fori_loop(0, _PPW // 2, step, 0)
    wwait(_PPW - 1, rows1, w1)


def kernel(input_ids, weight):
    mu_key = jax.random.key(42)
    mu = jax.random.randint(mu_key, (_P, _V, _D), 0, 2).astype(jnp.float32) * 2.0 - 1.0

    table = pl.pallas_call(
        _build_table_body,
        out_shape=jax.ShapeDtypeStruct((_P, _V, _D), jnp.float32),
    )(weight, mu)
    table2d = table.reshape(_P * _V, _D)

    ids = input_ids.astype(jnp.int32)
    gidx = ids[None, :, :] + _V * jnp.arange(_P, dtype=jnp.int32)[:, None, None]
    gidx = gidx.reshape(_NW, _PPW, _L)

    out = pl.kernel(
        _gather_body,
        out_type=jax.ShapeDtypeStruct((_P, _B, _L, _D), jnp.float32),
        mesh=plsc.VectorSubcoreMesh(core_axis_name="c", subcore_axis_name="s"),
        scratch_types=[
            pltpu.VMEM((_PPW, _L), jnp.int32),
            pltpu.VMEM((_L, _D), jnp.float32),
            pltpu.VMEM((_L, _D), jnp.float32),
            pltpu.SemaphoreType.DMA,
            pltpu.SemaphoreType.DMA,
            pltpu.SemaphoreType.DMA,
            pltpu.SemaphoreType.DMA,
        ],
    )(table2d, gidx)

    return out
